# baseline (speedup 1.0000x reference)
import jax
import jax.numpy as jnp
from jax import lax
from jax.experimental import pallas as pl

_K = 8192
_D = 32
_TN = 256


def _dist_argmin_body(z_ref, w_ref, zsq_ref, wsq_ref,
                      zq_ref, tok_ref, dmin_ref):
    zb = z_ref[0].reshape(_D, _TN)
    w = w_ref[...]
    mm = lax.dot_general(zb, w, (((0,), (1,)), ((), ())),
                         preferred_element_type=jnp.float32)
    zsqb = zsq_ref[0, 0]
    wsqb = wsq_ref[...]
    d = (zsqb + wsqb) - 2.0 * mm
    minv = jnp.min(d, axis=1, keepdims=True)
    idx = jnp.argmin(d, axis=1).astype(jnp.int32)[:, None]
    tok_ref[0, 0] = idx
    dmin_ref[0, 0] = minv
    iota = lax.broadcasted_iota(jnp.int32, d.shape, 1)
    onehot = (iota == idx).astype(jnp.float32)
    zq = lax.dot_general(onehot, w, (((1,), (0,)), ((), ())),
                         preferred_element_type=jnp.float32,
                         precision=lax.Precision.HIGHEST)
    zq_ref[0] = zq


def kernel(z, W):
    B, C, H, Wd = z.shape
    n_tok = B * H * Wd
    n_blk = n_tok // _TN
    q_per_b = n_blk // B
    h_per_q = H // q_per_b

    zt = jnp.transpose(z, (0, 2, 3, 1))
    zf = zt.reshape(-1, _D)
    zsq = jnp.sum(zf ** 2, axis=1).reshape(B, q_per_b, _TN, 1)
    wsq = jnp.sum(W ** 2, axis=1).reshape(1, _K)

    zq_rows, tok, dmin = pl.pallas_call(
        _dist_argmin_body,
        grid=(B, q_per_b),
        in_specs=[
            pl.BlockSpec((1, C, h_per_q, Wd), lambda b, q: (b, 0, q, 0)),
            pl.BlockSpec((_K, _D), lambda b, q: (0, 0)),
            pl.BlockSpec((1, 1, _TN, 1), lambda b, q: (b, q, 0, 0)),
            pl.BlockSpec((1, _K), lambda b, q: (0, 0)),
        ],
        out_specs=[
            pl.BlockSpec((1, _TN, _D), lambda b, q: (b * 4 + q, 0, 0)),
            pl.BlockSpec((1, 1, _TN, 1), lambda b, q: (b, q, 0, 0)),
            pl.BlockSpec((1, 1, _TN, 1), lambda b, q: (b, q, 0, 0)),
        ],
        out_shape=[
            jax.ShapeDtypeStruct((n_blk, _TN, _D), jnp.float32),
            jax.ShapeDtypeStruct((B, q_per_b, _TN, 1), jnp.int32),
            jax.ShapeDtypeStruct((B, q_per_b, _TN, 1), jnp.float32),
        ],
    )(z, W, zsq, wsq)

    token = tok.reshape(-1)
    z_q = zq_rows.reshape(zt.shape)
    z_q_st = zt + (z_q - zt)
    z_q_out = jnp.transpose(z_q_st, (0, 3, 1, 2))

    quant_error = jnp.mean(dmin)
    loss = 0.25 * (quant_error / _D)

    histogram = jnp.bincount(token, minlength=_K, length=_K).astype(jnp.float32)
    codebook_usage_counts = jnp.sum((histogram > 0).astype(jnp.float32))
    codebook_utilization = codebook_usage_counts / _K
    avg_probs = histogram / jnp.sum(histogram)
    codebook_perplexity = jnp.exp(-jnp.sum(avg_probs * jnp.log(avg_probs + 1e-10)))

    return (z_q_out, loss, quant_error, codebook_utilization, codebook_perplexity)
